# baseline (speedup 1.0000x reference)
import functools

import jax
import jax.numpy as jnp
from jax import lax
from jax.experimental import pallas as pl
from jax.experimental.pallas import tpu as pltpu
from jax.experimental.pallas import tpu_sc as plsc

_N = 48
_TABLE = 14
_LANES = 16
_ROWS_PER_W = 4
_NWORK = _N // _ROWS_PER_W

_mesh = plsc.VectorSubcoreMesh(
    core_axis_name="c", subcore_axis_name="s", num_cores=1
)


@functools.partial(
    pl.kernel,
    mesh=_mesh,
    out_type=jax.ShapeDtypeStruct((_N, _N), jnp.float32),
    scratch_types=[
        pltpu.VMEM((_TABLE,), jnp.float32),
        pltpu.VMEM((_ROWS_PER_W, _N), jnp.int32),
        pltpu.VMEM((_ROWS_PER_W, _N), jnp.float32),
    ],
    compiler_params=pltpu.CompilerParams(needs_layout_passes=False),
)
def _sc_embed_lookup(emb_hbm, idx_hbm, out_hbm, emb_v, idx_v, out_v):
    wid = lax.axis_index("s")

    @pl.when(wid < _NWORK)
    def _():
        base = wid * _ROWS_PER_W
        pltpu.sync_copy(emb_hbm, emb_v)
        pltpu.sync_copy(idx_hbm.at[pl.ds(base, _ROWS_PER_W), :], idx_v)

        def row(r, _):
            for j in range(_N // _LANES):
                iv = idx_v[r, pl.ds(j * _LANES, _LANES)]
                out_v[r, pl.ds(j * _LANES, _LANES)] = plsc.load_gather(
                    emb_v, [iv]
                )
            return _

        lax.fori_loop(0, _ROWS_PER_W, row, None)
        pltpu.sync_copy(out_v, out_hbm.at[pl.ds(base, _ROWS_PER_W), :])


@jax.jit
def kernel(dist_embed, dist_matrix):
    return _sc_embed_lookup(dist_embed, dist_matrix.astype(jnp.int32))


# speedup vs baseline: 1.0035x; 1.0035x over previous
import functools

import jax
import jax.numpy as jnp
from jax import lax
from jax.experimental import pallas as pl
from jax.experimental.pallas import tpu as pltpu
from jax.experimental.pallas import tpu_sc as plsc

_N = 48
_TABLE = 14
_LANES = 16
_ROWS_PER_W = 4
_NWORK = _N // _ROWS_PER_W

_mesh = plsc.VectorSubcoreMesh(
    core_axis_name="c", subcore_axis_name="s", num_cores=1
)


@functools.partial(
    pl.kernel,
    mesh=_mesh,
    out_type=jax.ShapeDtypeStruct((_N, _N), jnp.float32),
    scratch_types=[
        pltpu.VMEM((_TABLE,), jnp.float32),
        pltpu.VMEM((_ROWS_PER_W, _N), jnp.int32),
        pltpu.VMEM((_ROWS_PER_W, _N), jnp.float32),
    ],
    compiler_params=pltpu.CompilerParams(
        needs_layout_passes=False,
        skip_device_barrier=True,
        disable_bounds_checks=True,
        disable_semaphore_checks=True,
    ),
)
def _sc_embed_lookup(emb_hbm, idx_hbm, out_hbm, emb_v, idx_v, out_v):
    wid = lax.axis_index("s")

    @pl.when(wid < _NWORK)
    def _():
        base = wid * _ROWS_PER_W
        pltpu.sync_copy(emb_hbm, emb_v)
        pltpu.sync_copy(idx_hbm.at[pl.ds(base, _ROWS_PER_W), :], idx_v)

        def row(r, _):
            for j in range(_N // _LANES):
                iv = idx_v[r, pl.ds(j * _LANES, _LANES)]
                out_v[r, pl.ds(j * _LANES, _LANES)] = plsc.load_gather(
                    emb_v, [iv]
                )
            return _

        lax.fori_loop(0, _ROWS_PER_W, row, None)
        pltpu.sync_copy(out_v, out_hbm.at[pl.ds(base, _ROWS_PER_W), :])


@jax.jit
def kernel(dist_embed, dist_matrix):
    return _sc_embed_lookup(dist_embed, dist_matrix.astype(jnp.int32))


# speedup vs baseline: 1.0341x; 1.0304x over previous
import functools

import jax
import jax.numpy as jnp
from jax import lax
from jax.experimental import pallas as pl
from jax.experimental.pallas import tpu as pltpu
from jax.experimental.pallas import tpu_sc as plsc

_N = 48
_TABLE = 14
_LANES = 16
_ROWS_PER_W = 4
_NWORK = _N // _ROWS_PER_W

_mesh = plsc.VectorSubcoreMesh(
    core_axis_name="c", subcore_axis_name="s", num_cores=1
)


@functools.partial(
    pl.kernel,
    mesh=_mesh,
    out_type=jax.ShapeDtypeStruct((_N, _N), jnp.float32),
    scratch_types=[
        pltpu.VMEM((_TABLE,), jnp.float32),
        pltpu.VMEM((_ROWS_PER_W, _N), jnp.int32),
        pltpu.VMEM((_ROWS_PER_W, _N), jnp.float32),
        pltpu.SemaphoreType.DMA,
        pltpu.SemaphoreType.DMA,
    ],
    compiler_params=pltpu.CompilerParams(
        needs_layout_passes=False,
        skip_device_barrier=True,
        disable_bounds_checks=True,
        disable_semaphore_checks=True,
    ),
)
def _sc_embed_lookup(emb_hbm, idx_hbm, out_hbm, emb_v, idx_v, out_v, sem_e, sem_i):
    wid = lax.axis_index("s")

    @pl.when(wid < _NWORK)
    def _():
        base = wid * _ROWS_PER_W
        cp_e = pltpu.make_async_copy(emb_hbm, emb_v, sem_e)
        cp_i = pltpu.make_async_copy(
            idx_hbm.at[pl.ds(base, _ROWS_PER_W), :], idx_v, sem_i
        )
        cp_e.start()
        cp_i.start()
        cp_e.wait()
        cp_i.wait()

        def row(r, _):
            for j in range(_N // _LANES):
                iv = idx_v[r, pl.ds(j * _LANES, _LANES)]
                out_v[r, pl.ds(j * _LANES, _LANES)] = plsc.load_gather(
                    emb_v, [iv]
                )
            return _

        lax.fori_loop(0, _ROWS_PER_W, row, None)
        pltpu.sync_copy(out_v, out_hbm.at[pl.ds(base, _ROWS_PER_W), :])


@jax.jit
def kernel(dist_embed, dist_matrix):
    return _sc_embed_lookup(dist_embed, dist_matrix.astype(jnp.int32))


# speedup vs baseline: 13.1706x; 12.7367x over previous
import jax
import jax.numpy as jnp
from jax.experimental import pallas as pl
from jax.experimental.pallas import tpu as pltpu

_N = 48
_TABLE = 14


def _body(emb_ref, idx_ref, out_ref):
    idx = idx_ref[...]
    acc = jnp.full((_N, _N), 0.0, dtype=jnp.float32)
    for k in range(_TABLE):
        acc = jnp.where(idx == k, emb_ref[k], acc)
    out_ref[...] = acc


@jax.jit
def kernel(dist_embed, dist_matrix):
    return pl.pallas_call(
        _body,
        out_shape=jax.ShapeDtypeStruct((_N, _N), jnp.float32),
        in_specs=[
            pl.BlockSpec(memory_space=pltpu.SMEM),
            pl.BlockSpec(memory_space=pltpu.VMEM),
        ],
        out_specs=pl.BlockSpec(memory_space=pltpu.VMEM),
    )(dist_embed, dist_matrix.astype(jnp.int32))
